# baseline (speedup 1.0000x reference)
import functools

import jax
import jax.numpy as jnp
from jax import lax
from jax.experimental import pallas as pl
from jax.experimental.pallas import tpu as pltpu
from jax.experimental.pallas import tpu_sc as plsc

_NW = 32
_LANES = 16
_CH = 32
_NBUF = 4
_N_GRID = 512


def _sc_body(g0inv_hbm, zeros_hbm, params_hbm, out_hbm,
             g0inv_v, params_v, *rest):
    bufs = rest[0:_NBUF]
    idxs = rest[_NBUF:2 * _NBUF]
    sems = rest[2 * _NBUF:3 * _NBUF]
    cid = lax.axis_index("c")
    sid = lax.axis_index("s")
    wid = sid * 2 + cid
    rows_per_w = params_v.shape[0]
    n_chunks = rows_per_w // _CH
    row_base = wid * rows_per_w

    pltpu.sync_copy(params_hbm.at[pl.ds(row_base * 1, rows_per_w)], params_v)
    pltpu.sync_copy(g0inv_hbm, g0inv_v)
    for buf in bufs:
        pltpu.sync_copy(zeros_hbm, buf)

    g0 = g0inv_v[pl.ds(0, _LANES)]
    inv_dx = g0inv_v[pl.ds(_LANES, _LANES)]
    lane = jnp.arange(_LANES, dtype=jnp.int32)
    zeros16 = jnp.zeros((_LANES,), jnp.float32)

    copies = [None] * _NBUF

    for k in range(n_chunks):
        b = k % _NBUF
        buf, idxb, sem = bufs[b], idxs[b], sems[b]
        if copies[b] is not None:
            copies[b].wait()
            for j in range(_CH // _LANES):
                rvec = lane + (j * _LANES)
                cv = idxb[pl.ds(j * _LANES, _LANES)]
                plsc.store_scatter(buf, [rvec, cv], zeros16)
                plsc.store_scatter(buf, [rvec, cv + 1], zeros16)
        for j in range(_CH // _LANES):
            x = params_v[pl.ds(k * _CH + j * _LANES, _LANES)]
            t = (x - g0) * inv_dx
            ind = jnp.minimum(t.astype(jnp.int32) + 1, _N_GRID - 1)
            w0 = ind.astype(jnp.float32) - t
            w1 = 1.0 - w0
            rvec = lane + (j * _LANES)
            cv = ind - 1
            plsc.store_scatter(buf, [rvec, cv], w0)
            plsc.store_scatter(buf, [rvec, cv + 1], w1)
            idxb[pl.ds(j * _LANES, _LANES)] = cv
        copies[b] = pltpu.async_copy(
            buf, out_hbm.at[pl.ds(row_base + k * _CH, _CH)], sem)
    for cp in copies:
        cp.wait()


@jax.jit
def kernel(params, log_tau):
    n_rows = params.shape[0]
    n_grid = log_tau.shape[0]
    g0 = log_tau[0]
    dx = (log_tau[-1] - log_tau[0]) / (n_grid - 1)
    g0inv = jnp.concatenate(
        [jnp.full((_LANES,), g0), jnp.full((_LANES,), 1.0 / dx)])
    zeros_tile = jnp.zeros((_CH, n_grid), jnp.float32)
    rows_per_w = n_rows // _NW

    mesh = plsc.VectorSubcoreMesh(core_axis_name="c", subcore_axis_name="s")
    sc_call = functools.partial(
        pl.kernel,
        mesh=mesh,
        out_type=jax.ShapeDtypeStruct((n_rows, n_grid), jnp.float32),
        scratch_types=(
            [pltpu.VMEM((2 * _LANES,), jnp.float32),
             pltpu.VMEM((rows_per_w,), jnp.float32)]
            + [pltpu.VMEM((_CH, n_grid), jnp.float32)] * _NBUF
            + [pltpu.VMEM((_CH,), jnp.int32)] * _NBUF
            + [pltpu.SemaphoreType.DMA] * _NBUF
        ),
        compiler_params=pltpu.CompilerParams(needs_layout_passes=False),
    )(_sc_body)
    return sc_call(g0inv, zeros_tile, params.reshape(-1))


# speedup vs baseline: 1.0133x; 1.0133x over previous
import functools

import jax
import jax.numpy as jnp
from jax import lax
from jax.experimental import pallas as pl
from jax.experimental.pallas import tpu as pltpu
from jax.experimental.pallas import tpu_sc as plsc

_NW = 32
_LANES = 16
_CH = 64
_NBUF = 3
_N_GRID = 512


def _sc_body(g0inv_hbm, zeros_hbm, params_hbm, out_hbm,
             g0inv_v, params_v, *rest):
    bufs = rest[0:_NBUF]
    idxs = rest[_NBUF:2 * _NBUF]
    sems = rest[2 * _NBUF:3 * _NBUF]
    cid = lax.axis_index("c")
    sid = lax.axis_index("s")
    wid = sid * 2 + cid
    rows_per_w = params_v.shape[0]
    n_chunks = rows_per_w // _CH
    row_base = wid * rows_per_w

    pltpu.sync_copy(params_hbm.at[pl.ds(row_base * 1, rows_per_w)], params_v)
    pltpu.sync_copy(g0inv_hbm, g0inv_v)
    for buf in bufs:
        pltpu.sync_copy(zeros_hbm, buf)

    g0 = g0inv_v[pl.ds(0, _LANES)]
    inv_dx = g0inv_v[pl.ds(_LANES, _LANES)]
    lane = jnp.arange(_LANES, dtype=jnp.int32)
    zeros16 = jnp.zeros((_LANES,), jnp.float32)

    copies = [None] * _NBUF

    for k in range(n_chunks):
        b = k % _NBUF
        buf, idxb, sem = bufs[b], idxs[b], sems[b]
        if copies[b] is not None:
            copies[b].wait()
            for j in range(_CH // _LANES):
                rvec = lane + (j * _LANES)
                cv = idxb[pl.ds(j * _LANES, _LANES)]
                plsc.store_scatter(buf, [rvec, cv], zeros16)
                plsc.store_scatter(buf, [rvec, cv + 1], zeros16)
        for j in range(_CH // _LANES):
            x = params_v[pl.ds(k * _CH + j * _LANES, _LANES)]
            t = (x - g0) * inv_dx
            ind = jnp.minimum(t.astype(jnp.int32) + 1, _N_GRID - 1)
            w0 = ind.astype(jnp.float32) - t
            w1 = 1.0 - w0
            rvec = lane + (j * _LANES)
            cv = ind - 1
            plsc.store_scatter(buf, [rvec, cv], w0)
            plsc.store_scatter(buf, [rvec, cv + 1], w1)
            idxb[pl.ds(j * _LANES, _LANES)] = cv
        copies[b] = pltpu.async_copy(
            buf, out_hbm.at[pl.ds(row_base + k * _CH, _CH)], sem)
    for cp in copies:
        cp.wait()


@jax.jit
def kernel(params, log_tau):
    n_rows = params.shape[0]
    n_grid = log_tau.shape[0]
    g0 = log_tau[0]
    dx = (log_tau[-1] - log_tau[0]) / (n_grid - 1)
    g0inv = jnp.concatenate(
        [jnp.full((_LANES,), g0), jnp.full((_LANES,), 1.0 / dx)])
    zeros_tile = jnp.zeros((_CH, n_grid), jnp.float32)
    rows_per_w = n_rows // _NW

    mesh = plsc.VectorSubcoreMesh(core_axis_name="c", subcore_axis_name="s")
    sc_call = functools.partial(
        pl.kernel,
        mesh=mesh,
        out_type=jax.ShapeDtypeStruct((n_rows, n_grid), jnp.float32),
        scratch_types=(
            [pltpu.VMEM((2 * _LANES,), jnp.float32),
             pltpu.VMEM((rows_per_w,), jnp.float32)]
            + [pltpu.VMEM((_CH, n_grid), jnp.float32)] * _NBUF
            + [pltpu.VMEM((_CH,), jnp.int32)] * _NBUF
            + [pltpu.SemaphoreType.DMA] * _NBUF
        ),
        compiler_params=pltpu.CompilerParams(needs_layout_passes=False),
    )(_sc_body)
    return sc_call(g0inv, zeros_tile, params.reshape(-1))


# speedup vs baseline: 1.0456x; 1.0319x over previous
import functools

import jax
import jax.numpy as jnp
from jax import lax
from jax.experimental import pallas as pl
from jax.experimental.pallas import tpu as pltpu
from jax.experimental.pallas import tpu_sc as plsc

_NW = 32
_LANES = 16
_CH = 64
_NBUF = 2
_N_GRID = 512


def _sc_body(grid_hbm, zeros_hbm, params_hbm, out_hbm,
             grid_v, params_v, *rest):
    bufs = rest[0:_NBUF]
    idxs = rest[_NBUF:2 * _NBUF]
    sems = rest[2 * _NBUF:3 * _NBUF]
    psem = rest[3 * _NBUF]
    cid = lax.axis_index("c")
    sid = lax.axis_index("s")
    wid = sid * 2 + cid
    rows_per_w = params_v.shape[0]
    n_chunks = rows_per_w // _CH
    row_base = wid * rows_per_w

    pcopy = pltpu.async_copy(
        params_hbm.at[pl.ds(row_base * 1, rows_per_w)], params_v, psem)
    copies = [pltpu.async_copy(zeros_hbm, buf, sem)
              for buf, sem in zip(bufs, sems)]
    pltpu.sync_copy(grid_hbm, grid_v)

    g0 = plsc.load_gather(grid_v, [jnp.zeros((_LANES,), jnp.int32)])
    gN = plsc.load_gather(
        grid_v, [jnp.full((_LANES,), _N_GRID - 1, jnp.int32)])
    inv_dx = float(_N_GRID - 1) / (gN - g0)
    lane = jnp.arange(_LANES, dtype=jnp.int32)
    zeros16 = jnp.zeros((_LANES,), jnp.float32)
    pcopy.wait()

    for k in range(n_chunks):
        b = k % _NBUF
        buf, idxb, sem = bufs[b], idxs[b], sems[b]
        copies[b].wait()
        if k >= _NBUF:
            for j in range(_CH // _LANES):
                rvec = lane + (j * _LANES)
                cv = idxb[pl.ds(j * _LANES, _LANES)]
                plsc.store_scatter(buf, [rvec, cv], zeros16)
                plsc.store_scatter(buf, [rvec, cv + 1], zeros16)
        for j in range(_CH // _LANES):
            x = params_v[pl.ds(k * _CH + j * _LANES, _LANES)]
            t = (x - g0) * inv_dx
            ind = jnp.minimum(t.astype(jnp.int32) + 1, _N_GRID - 1)
            w0 = ind.astype(jnp.float32) - t
            w1 = 1.0 - w0
            rvec = lane + (j * _LANES)
            cv = ind - 1
            plsc.store_scatter(buf, [rvec, cv], w0)
            plsc.store_scatter(buf, [rvec, cv + 1], w1)
            idxb[pl.ds(j * _LANES, _LANES)] = cv
        copies[b] = pltpu.async_copy(
            buf, out_hbm.at[pl.ds(row_base + k * _CH, _CH)], sem)
    for cp in copies:
        cp.wait()


@jax.jit
def kernel(params, log_tau):
    n_rows = params.shape[0]
    n_grid = log_tau.shape[0]
    zeros_tile = jnp.zeros((_CH, n_grid), jnp.float32)
    rows_per_w = n_rows // _NW

    mesh = plsc.VectorSubcoreMesh(core_axis_name="c", subcore_axis_name="s")
    sc_call = functools.partial(
        pl.kernel,
        mesh=mesh,
        out_type=jax.ShapeDtypeStruct((n_rows, n_grid), jnp.float32),
        scratch_types=(
            [pltpu.VMEM((n_grid,), jnp.float32),
             pltpu.VMEM((rows_per_w,), jnp.float32)]
            + [pltpu.VMEM((_CH, n_grid), jnp.float32)] * _NBUF
            + [pltpu.VMEM((_CH,), jnp.int32)] * _NBUF
            + [pltpu.SemaphoreType.DMA] * _NBUF
            + [pltpu.SemaphoreType.DMA]
        ),
        compiler_params=pltpu.CompilerParams(needs_layout_passes=False),
    )(_sc_body)
    return sc_call(log_tau, zeros_tile, params.reshape(-1))


# speedup vs baseline: 1.0645x; 1.0180x over previous
import functools

import jax
import jax.numpy as jnp
from jax import lax
from jax.experimental import pallas as pl
from jax.experimental.pallas import tpu as pltpu
from jax.experimental.pallas import tpu_sc as plsc

_NW = 32
_LANES = 16
_CH = 64
_NBUF = 2
_N_GRID = 512


def _sc_body(g0inv_hbm, zeros_hbm, params_hbm, out_hbm,
             g0inv_v, params_v, *rest):
    bufs = rest[0:_NBUF]
    idxs = rest[_NBUF:2 * _NBUF]
    sems = rest[2 * _NBUF:3 * _NBUF]
    cid = lax.axis_index("c")
    sid = lax.axis_index("s")
    wid = sid * 2 + cid
    rows_per_w = params_v.shape[0]
    n_chunks = rows_per_w // _CH
    row_base = wid * rows_per_w

    pltpu.sync_copy(params_hbm.at[pl.ds(row_base * 1, rows_per_w)], params_v)
    pltpu.sync_copy(g0inv_hbm, g0inv_v)
    for buf in bufs:
        pltpu.sync_copy(zeros_hbm, buf)

    g0 = g0inv_v[pl.ds(0, _LANES)]
    inv_dx = g0inv_v[pl.ds(_LANES, _LANES)]
    lane = jnp.arange(_LANES, dtype=jnp.int32)
    zeros16 = jnp.zeros((_LANES,), jnp.float32)

    copies = [None] * _NBUF

    for k in range(n_chunks):
        b = k % _NBUF
        buf, idxb, sem = bufs[b], idxs[b], sems[b]
        if copies[b] is not None:
            copies[b].wait()
            for j in range(_CH // _LANES):
                rvec = lane + (j * _LANES)
                cv = idxb[pl.ds(j * _LANES, _LANES)]
                plsc.store_scatter(buf, [rvec, cv], zeros16)
                plsc.store_scatter(buf, [rvec, cv + 1], zeros16)
        for j in range(_CH // _LANES):
            x = params_v[pl.ds(k * _CH + j * _LANES, _LANES)]
            t = (x - g0) * inv_dx
            ind = jnp.minimum(t.astype(jnp.int32) + 1, _N_GRID - 1)
            w0 = ind.astype(jnp.float32) - t
            w1 = 1.0 - w0
            rvec = lane + (j * _LANES)
            cv = ind - 1
            plsc.store_scatter(buf, [rvec, cv], w0)
            plsc.store_scatter(buf, [rvec, cv + 1], w1)
            idxb[pl.ds(j * _LANES, _LANES)] = cv
        copies[b] = pltpu.async_copy(
            buf, out_hbm.at[pl.ds(row_base + k * _CH, _CH)], sem)
    for cp in copies:
        cp.wait()


@jax.jit
def kernel(params, log_tau):
    n_rows = params.shape[0]
    n_grid = log_tau.shape[0]
    g0 = log_tau[0]
    dx = (log_tau[-1] - log_tau[0]) / (n_grid - 1)
    g0inv = jnp.concatenate(
        [jnp.full((_LANES,), g0), jnp.full((_LANES,), 1.0 / dx)])
    zeros_tile = jnp.zeros((_CH, n_grid), jnp.float32)
    rows_per_w = n_rows // _NW

    mesh = plsc.VectorSubcoreMesh(core_axis_name="c", subcore_axis_name="s")
    sc_call = functools.partial(
        pl.kernel,
        mesh=mesh,
        out_type=jax.ShapeDtypeStruct((n_rows, n_grid), jnp.float32),
        scratch_types=(
            [pltpu.VMEM((2 * _LANES,), jnp.float32),
             pltpu.VMEM((rows_per_w,), jnp.float32)]
            + [pltpu.VMEM((_CH, n_grid), jnp.float32)] * _NBUF
            + [pltpu.VMEM((_CH,), jnp.int32)] * _NBUF
            + [pltpu.SemaphoreType.DMA] * _NBUF
        ),
        compiler_params=pltpu.CompilerParams(needs_layout_passes=False),
    )(_sc_body)
    return sc_call(g0inv, zeros_tile, params.reshape(-1))
